# baseline (speedup 1.0000x reference)
import functools

import jax
import jax.numpy as jnp
from jax import lax
from jax.experimental import pallas as pl
from jax.experimental.pallas import tpu as pltpu
from jax.experimental.pallas import tpu_sc as plsc

_BOUNDS = (-2.0, -1.5, -1.0, -0.5, 0.0, 0.5, 1.0, 1.5, 2.0)

_ROWS, _COLS = 16384, 26
_NC, _NS, _L = 2, 16, 16
_NW = _NC * _NS
_ROWS_W = _ROWS // _NW
_CHUNK = 128
_UNROLL = 8

_mesh = plsc.VectorSubcoreMesh(core_axis_name="c", subcore_axis_name="s")


@functools.partial(
    pl.kernel,
    mesh=_mesh,
    out_type=jax.ShapeDtypeStruct((_ROWS, _COLS), jnp.int32),
    scratch_types=[
        pltpu.VMEM((_CHUNK, _COLS), jnp.float32),
        pltpu.VMEM((_CHUNK, _COLS), jnp.int32),
    ],
)
def _bucketize_sc(x_hbm, out_hbm, x_v, o_v):
    wid = lax.axis_index("s") * _NC + lax.axis_index("c")
    r0 = wid * _ROWS_W

    bvecs = [jnp.full((_L,), b, jnp.float32) for b in _BOUNDS]
    one = jnp.ones((_L,), jnp.int32)
    zero = jnp.zeros((_L,), jnp.int32)

    def bucketize(x):
        acc = zero
        for bv in bvecs:
            acc = acc + jnp.where(x > bv, one, zero)
        return acc

    def chunk(c, carry):
        base = r0 + c * _CHUNK
        pltpu.sync_copy(x_hbm.at[pl.ds(base, _CHUNK)], x_v)

        def body(i, carry2):
            for u in range(_UNROLL):
                r = i * _UNROLL + u
                o_v[r, pl.ds(0, _L)] = bucketize(x_v[r, pl.ds(0, _L)])
                o_v[r, pl.ds(_COLS - _L, _L)] = bucketize(
                    x_v[r, pl.ds(_COLS - _L, _L)])
            return carry2

        lax.fori_loop(0, _CHUNK // _UNROLL, body, 0)
        pltpu.sync_copy(o_v, out_hbm.at[pl.ds(base, _CHUNK)])
        return carry

    lax.fori_loop(0, _ROWS_W // _CHUNK, chunk, 0)


def kernel(inputs):
    return _bucketize_sc(inputs)


# speedup vs baseline: 1.8579x; 1.8579x over previous
import functools

import jax
import jax.numpy as jnp
from jax import lax
from jax.experimental import pallas as pl
from jax.experimental.pallas import tpu as pltpu
from jax.experimental.pallas import tpu_sc as plsc

_BOUNDS = (-2.0, -1.5, -1.0, -0.5, 0.0, 0.5, 1.0, 1.5, 2.0)

_ROWS, _COLS = 16384, 26
_NC, _NS, _L = 2, 16, 16
_NW = _NC * _NS
_ROWS_W = _ROWS // _NW
_CHUNK = 128
_UNROLL = 1

_mesh = plsc.VectorSubcoreMesh(core_axis_name="c", subcore_axis_name="s")


@functools.partial(
    pl.kernel,
    mesh=_mesh,
    out_type=jax.ShapeDtypeStruct((_ROWS, _COLS), jnp.int32),
    scratch_types=[
        pltpu.VMEM((_CHUNK, _COLS), jnp.float32),
        pltpu.VMEM((_CHUNK, _COLS), jnp.int32),
    ],
    compiler_params=pltpu.CompilerParams(use_tc_tiling_on_sc=True),
)
def _bucketize_sc(x_hbm, out_hbm, x_v, o_v):
    wid = lax.axis_index("s") * _NC + lax.axis_index("c")
    r0 = wid * _ROWS_W

    bvecs = [jnp.full((_L,), b, jnp.float32) for b in _BOUNDS]
    one = jnp.ones((_L,), jnp.int32)
    zero = jnp.zeros((_L,), jnp.int32)

    def bucketize(x):
        acc = zero
        for bv in bvecs:
            acc = acc + jnp.where(x > bv, one, zero)
        return acc

    def chunk(c, carry):
        base = r0 + c * _CHUNK
        pltpu.sync_copy(x_hbm.at[pl.ds(base, _CHUNK)], x_v)

        def body(i, carry2):
            for u in range(_UNROLL):
                r = i * _UNROLL + u
                o_v[r, pl.ds(0, _L)] = bucketize(x_v[r, pl.ds(0, _L)])
                o_v[r, pl.ds(_COLS - _L, _L)] = bucketize(
                    x_v[r, pl.ds(_COLS - _L, _L)])
            return carry2

        lax.fori_loop(0, _CHUNK // _UNROLL, body, 0)
        pltpu.sync_copy(o_v, out_hbm.at[pl.ds(base, _CHUNK)])
        return carry

    lax.fori_loop(0, _ROWS_W // _CHUNK, chunk, 0)


def kernel(inputs):
    return _bucketize_sc(inputs)


# speedup vs baseline: 1.8839x; 1.0140x over previous
import functools

import jax
import jax.numpy as jnp
from jax import lax
from jax.experimental import pallas as pl
from jax.experimental.pallas import tpu as pltpu
from jax.experimental.pallas import tpu_sc as plsc

_BOUNDS = (-2.0, -1.5, -1.0, -0.5, 0.0, 0.5, 1.0, 1.5, 2.0)

_ROWS, _COLS = 16384, 26
_NC, _NS, _L = 2, 16, 16
_NW = _NC * _NS
_ROWS_W = _ROWS // _NW
_CHUNK = 128
_UNROLL = 4

_mesh = plsc.VectorSubcoreMesh(core_axis_name="c", subcore_axis_name="s")


@functools.partial(
    pl.kernel,
    mesh=_mesh,
    out_type=jax.ShapeDtypeStruct((_ROWS, _COLS), jnp.int32),
    scratch_types=[
        pltpu.VMEM((_CHUNK, _COLS), jnp.float32),
        pltpu.VMEM((_CHUNK, _COLS), jnp.int32),
    ],
    compiler_params=pltpu.CompilerParams(use_tc_tiling_on_sc=True),
)
def _bucketize_sc(x_hbm, out_hbm, x_v, o_v):
    wid = lax.axis_index("s") * _NC + lax.axis_index("c")
    r0 = wid * _ROWS_W

    bvecs = [jnp.full((_L,), b, jnp.float32) for b in _BOUNDS]
    one = jnp.ones((_L,), jnp.int32)
    zero = jnp.zeros((_L,), jnp.int32)

    def bucketize(x):
        acc = zero
        for bv in bvecs:
            acc = acc + jnp.where(x > bv, one, zero)
        return acc

    def chunk(c, carry):
        base = r0 + c * _CHUNK
        pltpu.sync_copy(x_hbm.at[pl.ds(base, _CHUNK)], x_v)

        @plsc.parallel_loop(0, _CHUNK, step=1, unroll=_UNROLL)
        def body(r):
            o_v[r, pl.ds(0, _L)] = bucketize(x_v[r, pl.ds(0, _L)])
            o_v[r, pl.ds(_COLS - _L, _L)] = bucketize(
                x_v[r, pl.ds(_COLS - _L, _L)])

        pltpu.sync_copy(o_v, out_hbm.at[pl.ds(base, _CHUNK)])
        return carry

    lax.fori_loop(0, _ROWS_W // _CHUNK, chunk, 0)


def kernel(inputs):
    return _bucketize_sc(inputs)


# speedup vs baseline: 2.8133x; 1.4933x over previous
import functools

import jax
import jax.numpy as jnp
from jax import lax
from jax.experimental import pallas as pl
from jax.experimental.pallas import tpu as pltpu
from jax.experimental.pallas import tpu_sc as plsc

_BOUNDS = (-2.0, -1.5, -1.0, -0.5, 0.0, 0.5, 1.0, 1.5, 2.0)

_ROWS, _COLS = 16384, 26
_NC, _NS, _L = 2, 16, 16
_NW = _NC * _NS
_COLS_W = _ROWS // _NW
_VECS = _COLS_W // _L

_mesh = plsc.VectorSubcoreMesh(core_axis_name="c", subcore_axis_name="s")


@functools.partial(
    pl.kernel,
    mesh=_mesh,
    out_type=jax.ShapeDtypeStruct((_COLS, _ROWS), jnp.int32),
    scratch_types=[
        pltpu.VMEM((_COLS, _COLS_W), jnp.float32),
        pltpu.VMEM((_COLS, _COLS_W), jnp.int32),
    ],
    compiler_params=pltpu.CompilerParams(use_tc_tiling_on_sc=True),
)
def _bucketize_sc(x_hbm, out_hbm, x_v, o_v):
    wid = lax.axis_index("s") * _NC + lax.axis_index("c")
    c0 = wid * _COLS_W
    pltpu.sync_copy(x_hbm.at[:, pl.ds(c0, _COLS_W)], x_v)

    bvecs = [jnp.full((_L,), b, jnp.float32) for b in _BOUNDS]
    one = jnp.ones((_L,), jnp.int32)
    zero = jnp.zeros((_L,), jnp.int32)

    def bucketize(x):
        acc = zero
        for bv in bvecs:
            acc = acc + jnp.where(x > bv, one, zero)
        return acc

    @plsc.parallel_loop(0, _VECS, step=1)
    def body(v):
        base = v * _L
        for r in range(_COLS):
            o_v[r, pl.ds(base, _L)] = bucketize(x_v[r, pl.ds(base, _L)])

    pltpu.sync_copy(o_v, out_hbm.at[:, pl.ds(c0, _COLS_W)])


def kernel(inputs):
    return _bucketize_sc(inputs.T).T


# speedup vs baseline: 2.8183x; 1.0018x over previous
import functools

import jax
import jax.numpy as jnp
from jax import lax
from jax.experimental import pallas as pl
from jax.experimental.pallas import tpu as pltpu
from jax.experimental.pallas import tpu_sc as plsc

_BOUNDS = (-2.0, -1.5, -1.0, -0.5, 0.0, 0.5, 1.0, 1.5, 2.0)

_ROWS, _COLS = 16384, 26
_NC, _NS, _L = 2, 16, 16
_NW = _NC * _NS
_COLS_W = _ROWS // _NW
_VECS = _COLS_W // _L

_mesh = plsc.VectorSubcoreMesh(core_axis_name="c", subcore_axis_name="s")


@functools.partial(
    pl.kernel,
    mesh=_mesh,
    out_type=jax.ShapeDtypeStruct((_COLS, _ROWS), jnp.int32),
    scratch_types=[
        pltpu.VMEM((_COLS, _COLS_W), jnp.float32),
        pltpu.VMEM((_COLS, _COLS_W), jnp.int32),
    ],
    compiler_params=pltpu.CompilerParams(
        use_tc_tiling_on_sc=True, skip_device_barrier=True),
)
def _bucketize_sc(x_hbm, out_hbm, x_v, o_v):
    wid = lax.axis_index("s") * _NC + lax.axis_index("c")
    c0 = wid * _COLS_W
    pltpu.sync_copy(x_hbm.at[:, pl.ds(c0, _COLS_W)], x_v)

    bvecs = [jnp.full((_L,), b, jnp.float32) for b in _BOUNDS]
    one = jnp.ones((_L,), jnp.int32)
    zero = jnp.zeros((_L,), jnp.int32)

    def bucketize(x):
        acc = zero
        for bv in bvecs:
            acc = acc + jnp.where(x > bv, one, zero)
        return acc

    @plsc.parallel_loop(0, _VECS, step=1)
    def body(v):
        base = v * _L
        for r in range(_COLS):
            o_v[r, pl.ds(base, _L)] = bucketize(x_v[r, pl.ds(base, _L)])

    pltpu.sync_copy(o_v, out_hbm.at[:, pl.ds(c0, _COLS_W)])


def kernel(inputs):
    return _bucketize_sc(inputs.T).T
